# baseline (speedup 1.0000x reference)
import functools

import jax
import jax.numpy as jnp
from jax import lax
from jax.experimental import pallas as pl
from jax.experimental.pallas import tpu as pltpu
from jax.experimental.pallas import tpu_sc as plsc

_B = 262144
_NC, _NS, _L = 2, 16, 16
_NW = _NC * _NS
_EPW = _B // _NW
_CH = 1024
_NSUB = _EPW // _CH
_NBT = _B // 128

_PT_SIZE = (8, 64, 512, 4096)
_POS_SIZE = (64, 512, 4096, 4096)
_PT_OFF = []
_POS_OFF = []
_off = 0
for _n in range(4):
    _PT_OFF.append(_off)
    _off += 3 * _PT_SIZE[_n]
for _n in range(4):
    _POS_OFF.append(_off)
    _off += 5 * _POS_SIZE[_n]
_TABF_LEN = _off
_H_OFF = 5 * 4096
_TABI_LEN = _H_OFF + 5 * 8


def _sc_body(tb_hbm, tabf_hbm, tabi_hbm, out_hbm, tabf, tabi, bitsv, outv,
             sem_in, sem_out):
    wid = lax.axis_index("s") * _NC + lax.axis_index("c")
    pltpu.sync_copy(tabf_hbm, tabf)
    pltpu.sync_copy(tabi_hbm, tabi)

    for s in range(_NSUB):
        base = wid * _EPW + s * _CH
        handles = []
        for k in range(12):
            src = tb_hbm.at[pl.ds(pl.multiple_of(k * _B + base, 8), _CH)]
            dst = bitsv.at[pl.ds(k * _CH, _CH)]
            handles.append(pltpu.async_copy(src, dst, sem_in))
        for h in handles:
            h.wait()

        @plsc.parallel_loop(0, _CH // _L, 1, unroll=8)
        def vec_body(v):
            e = v * _L
            bs = [bitsv[pl.ds(k * _CH + e, _L)] for k in range(12)]
            pairs = [bs[k] * 2 + bs[k + 1] for k in range(0, 12, 2)]
            quads = [pairs[i] * 4 + pairs[i + 1] for i in range(0, 6, 2)]
            addr = (quads[0] * 16 + quads[1]) * 16 + quads[2]
            ob = (v // 8) * 1024 + (v % 8) * _L
            for n in range(4):
                size = _PT_SIZE[n]
                an = jnp.bitwise_and(addr, size - 1) if n < 3 else addr
                pt_base = _PT_OFF[n] + an
                t0 = plsc.load_gather(tabf, [pt_base])
                t1 = plsc.load_gather(tabf, [pt_base + size])
                t2 = plsc.load_gather(tabf, [pt_base + 2 * size])
                h0 = (t0 > 0.5).astype(jnp.int32)
                h1 = (t1 > 0.5).astype(jnp.int32)
                h2 = (t2 > 0.5).astype(jnp.int32)
                hard = h0 * 4 + h1 * 2 + h2
                nb = n * (8 * _CH) + ob
                outv[pl.ds(nb, _L)] = t0
                outv[pl.ds(nb + 128, _L)] = t1
                outv[pl.ds(nb + 256, _L)] = t2
                if n < 3:
                    ap = _POS_OFF[n] + an * 8 + hard
                    psize = _POS_SIZE[n]
                    for j in range(5):
                        pj = plsc.load_gather(tabf, [ap + j * psize])
                        outv[pl.ds(nb + (3 + j) * 128, _L)] = pj
                else:
                    for j in range(5):
                        av = plsc.load_gather(tabi, [an + j * 4096])
                        hv = plsc.load_gather(tabi, [_H_OFF + j * 8 + hard])
                        pj = plsc.load_gather(tabf, [_POS_OFF[3] + av + hv])
                        outv[pl.ds(nb + (3 + j) * 128, _L)] = pj

        tb0 = base // 128
        handles = []
        for n in range(4):
            src = outv.at[pl.ds(n * (8 * _CH), 8 * _CH)]
            dst = out_hbm.at[pl.ds(
                pl.multiple_of(n * (_NBT * 1024) + tb0 * 1024, 8), 8 * _CH)]
            handles.append(pltpu.async_copy(src, dst, sem_out))
        for h in handles:
            h.wait()


@functools.cache
def _build_sc_forward():
    mesh = plsc.VectorSubcoreMesh(
        core_axis_name="c", subcore_axis_name="s",
        num_cores=_NC, num_subcores=_NS)
    return pl.kernel(
        _sc_body,
        out_type=jax.ShapeDtypeStruct((4 * _NBT * 1024,), jnp.float32),
        mesh=mesh,
        scratch_types=[
            pltpu.VMEM((_TABF_LEN,), jnp.float32),
            pltpu.VMEM((_TABI_LEN,), jnp.int32),
            pltpu.VMEM((_CH * 12,), jnp.int32),
            pltpu.VMEM((_CH * 32,), jnp.float32),
            pltpu.SemaphoreType.DMA,
            pltpu.SemaphoreType.DMA,
        ],
        compiler_params=pltpu.CompilerParams(needs_layout_passes=False),
    )


def _aux_tables(conn4):
    c = conn4.astype(jnp.int32)
    w = (jnp.int32(1) << (11 - jnp.arange(12, dtype=jnp.int32)))
    ai = jnp.arange(4096, dtype=jnp.int32)
    sa = jnp.clip(11 - c, 0, 31)
    bits_a = (ai[None, None, :] >> sa[:, :, None]) & 1
    a_tab = jnp.sum(
        jnp.where((c < 12)[:, :, None], bits_a, 0) * w[None, :, None], axis=1)
    a_tab = a_tab.astype(jnp.int32) + (jnp.arange(5, dtype=jnp.int32) * 4096)[:, None]
    hi = jnp.arange(8, dtype=jnp.int32)
    sh = jnp.clip(14 - c, 0, 31)
    bits_h = (hi[None, None, :] >> sh[:, :, None]) & 1
    h_tab = jnp.sum(
        jnp.where((c >= 12)[:, :, None], bits_h, 0) * w[None, :, None],
        axis=1).astype(jnp.int32)
    return a_tab, h_tab


def kernel(type_bits, pattern_table_1, pattern_table_2, pattern_table_3,
           pattern_table_4, position_table_1, position_table_2,
           position_table_3, position_table_4, conn4):
    assert type_bits.shape == (_B, 12)
    pts = [pattern_table_1, pattern_table_2, pattern_table_3, pattern_table_4]
    poss = [position_table_1, position_table_2, position_table_3,
            position_table_4]
    tabf = jnp.concatenate([p.reshape(-1) for p in pts]
                           + [p.reshape(-1) for p in poss])
    a_tab, h_tab = _aux_tables(conn4)
    tabi = jnp.concatenate([a_tab.reshape(-1), h_tab.reshape(-1)])
    tbp = jnp.transpose(type_bits).reshape(-1)
    out = _build_sc_forward()(tbp, tabf, tabi)
    return (out.reshape(4, _NBT, 8, 128)
            .transpose(1, 3, 0, 2).reshape(_B, 4, 8))


# speedup vs baseline: 1.1747x; 1.1747x over previous
import functools

import jax
import jax.numpy as jnp
from jax import lax
from jax.experimental import pallas as pl
from jax.experimental.pallas import tpu as pltpu
from jax.experimental.pallas import tpu_sc as plsc

_B = 262144
_NC, _NS, _L = 2, 16, 16
_NW = _NC * _NS
_EPW = _B // _NW
_CH = 512
_NSUB = _EPW // _CH
_NBT = _B // 128

_PT_SIZE = (8, 64, 512, 4096)
_POS_SIZE = (64, 512, 4096, 4096)
_PT_OFF = []
_POS_OFF = []
_off = 0
for _n in range(4):
    _PT_OFF.append(_off)
    _off += 3 * _PT_SIZE[_n]
for _n in range(4):
    _POS_OFF.append(_off)
    _off += 5 * _POS_SIZE[_n]
_TABF_LEN = _off
_H_OFF = 5 * 4096
_TABI_LEN = _H_OFF + 5 * 8


def _sc_body(tb_hbm, tabf_hbm, tabi_hbm, out_hbm, tabf, tabi, bitsv, outv,
             sem_in, sem_out):
    wid = lax.axis_index("s") * _NC + lax.axis_index("c")
    pltpu.sync_copy(tabf_hbm, tabf)
    pltpu.sync_copy(tabi_hbm, tabi)

    def fire_in(s, half):
        base = wid * _EPW + s * _CH
        hs = []
        for k in range(12):
            src = tb_hbm.at[pl.ds(pl.multiple_of(k * _B + base, 8), _CH)]
            dst = bitsv.at[pl.ds(half * (12 * _CH) + k * _CH, _CH)]
            hs.append(pltpu.async_copy(src, dst, sem_in.at[half]))
        return hs

    def fire_out(s, half):
        tb0 = (wid * _EPW + s * _CH) // 128
        hs = []
        for n in range(4):
            src = outv.at[pl.ds(half * (32 * _CH) + n * (8 * _CH), 8 * _CH)]
            dst = out_hbm.at[pl.ds(
                pl.multiple_of(n * (_NBT * 1024) + tb0 * 1024, 8), 8 * _CH)]
            hs.append(pltpu.async_copy(src, dst, sem_out.at[half]))
        return hs

    in_h = {0: fire_in(0, 0)}
    out_h = {}
    for s in range(_NSUB):
        half = s % 2
        for h in in_h.pop(s):
            h.wait()
        if s + 1 < _NSUB:
            in_h[s + 1] = fire_in(s + 1, 1 - half)
        if s - 2 in out_h:
            for h in out_h.pop(s - 2):
                h.wait()
        bb = half * (12 * _CH)
        ob0 = half * (32 * _CH)

        @plsc.parallel_loop(0, _CH // _L, 1, unroll=4)
        def vec_body(v):
            e = v * _L
            bs = [bitsv[pl.ds(bb + k * _CH + e, _L)] for k in range(12)]
            pairs = [bs[k] * 2 + bs[k + 1] for k in range(0, 12, 2)]
            quads = [pairs[i] * 4 + pairs[i + 1] for i in range(0, 6, 2)]
            addr = (quads[0] * 16 + quads[1]) * 16 + quads[2]
            ob = (v // 8) * 1024 + (v % 8) * _L
            for n in range(4):
                size = _PT_SIZE[n]
                an = jnp.bitwise_and(addr, size - 1) if n < 3 else addr
                pt_base = _PT_OFF[n] + an
                t0 = plsc.load_gather(tabf, [pt_base])
                t1 = plsc.load_gather(tabf, [pt_base + size])
                t2 = plsc.load_gather(tabf, [pt_base + 2 * size])
                h0 = (t0 > 0.5).astype(jnp.int32)
                h1 = (t1 > 0.5).astype(jnp.int32)
                h2 = (t2 > 0.5).astype(jnp.int32)
                hard = h0 * 4 + h1 * 2 + h2
                nb = ob0 + n * (8 * _CH) + ob
                outv[pl.ds(nb, _L)] = t0
                outv[pl.ds(nb + 128, _L)] = t1
                outv[pl.ds(nb + 256, _L)] = t2
                if n < 3:
                    ap = _POS_OFF[n] + an * 8 + hard
                    psize = _POS_SIZE[n]
                    for j in range(5):
                        pj = plsc.load_gather(tabf, [ap + j * psize])
                        outv[pl.ds(nb + (3 + j) * 128, _L)] = pj
                else:
                    for j in range(5):
                        av = plsc.load_gather(tabi, [an + j * 4096])
                        hv = plsc.load_gather(tabi, [_H_OFF + j * 8 + hard])
                        pj = plsc.load_gather(tabf, [_POS_OFF[3] + av + hv])
                        outv[pl.ds(nb + (3 + j) * 128, _L)] = pj

        out_h[s] = fire_out(s, half)

    for s in sorted(out_h):
        for h in out_h[s]:
            h.wait()


@functools.cache
def _build_sc_forward():
    mesh = plsc.VectorSubcoreMesh(
        core_axis_name="c", subcore_axis_name="s",
        num_cores=_NC, num_subcores=_NS)
    return pl.kernel(
        _sc_body,
        out_type=jax.ShapeDtypeStruct((4 * _NBT * 1024,), jnp.float32),
        mesh=mesh,
        scratch_types=[
            pltpu.VMEM((_TABF_LEN,), jnp.float32),
            pltpu.VMEM((_TABI_LEN,), jnp.int32),
            pltpu.VMEM((2 * _CH * 12,), jnp.int32),
            pltpu.VMEM((2 * _CH * 32,), jnp.float32),
            pltpu.SemaphoreType.DMA((2,)),
            pltpu.SemaphoreType.DMA((2,)),
        ],
        compiler_params=pltpu.CompilerParams(needs_layout_passes=False),
    )


def _aux_tables(conn4):
    c = conn4.astype(jnp.int32)
    w = (jnp.int32(1) << (11 - jnp.arange(12, dtype=jnp.int32)))
    ai = jnp.arange(4096, dtype=jnp.int32)
    sa = jnp.clip(11 - c, 0, 31)
    bits_a = (ai[None, None, :] >> sa[:, :, None]) & 1
    a_tab = jnp.sum(
        jnp.where((c < 12)[:, :, None], bits_a, 0) * w[None, :, None], axis=1)
    a_tab = a_tab.astype(jnp.int32) + (jnp.arange(5, dtype=jnp.int32) * 4096)[:, None]
    hi = jnp.arange(8, dtype=jnp.int32)
    sh = jnp.clip(14 - c, 0, 31)
    bits_h = (hi[None, None, :] >> sh[:, :, None]) & 1
    h_tab = jnp.sum(
        jnp.where((c >= 12)[:, :, None], bits_h, 0) * w[None, :, None],
        axis=1).astype(jnp.int32)
    return a_tab, h_tab


def kernel(type_bits, pattern_table_1, pattern_table_2, pattern_table_3,
           pattern_table_4, position_table_1, position_table_2,
           position_table_3, position_table_4, conn4):
    assert type_bits.shape == (_B, 12)
    pts = [pattern_table_1, pattern_table_2, pattern_table_3, pattern_table_4]
    poss = [position_table_1, position_table_2, position_table_3,
            position_table_4]
    tabf = jnp.concatenate([p.reshape(-1) for p in pts]
                           + [p.reshape(-1) for p in poss])
    a_tab, h_tab = _aux_tables(conn4)
    tabi = jnp.concatenate([a_tab.reshape(-1), h_tab.reshape(-1)])
    tbp = jnp.transpose(type_bits).reshape(-1)
    out = _build_sc_forward()(tbp, tabf, tabi)
    return (out.reshape(4, _NBT, 8, 128)
            .transpose(1, 3, 0, 2).reshape(_B, 4, 8))


# speedup vs baseline: 1.4351x; 1.2217x over previous
import functools

import jax
import jax.numpy as jnp
from jax import lax
from jax.experimental import pallas as pl
from jax.experimental.pallas import tpu as pltpu
from jax.experimental.pallas import tpu_sc as plsc

_B = 262144
_NC, _NS, _L = 2, 16, 16
_NW = _NC * _NS
_EPW = _B // _NW
_CH = 512
_NSUB = _EPW // _CH
_NBT = _B // 128

_PT_SIZE = (8, 64, 512, 4096)
_POS_SIZE = (64, 512, 4096, 4096)
_PT_OFF = []
_POS_OFF = []
_off = 0
for _n in range(4):
    _PT_OFF.append(_off)
    _off += 3 * _PT_SIZE[_n]
for _n in range(4):
    _POS_OFF.append(_off)
    _off += 5 * _POS_SIZE[_n]
_TABF_LEN = _off
_H_OFF = 5 * 4096
_TABI_LEN = _H_OFF + 5 * 8


def _sc_body(tb_hbm, tabf_hbm, tabi_hbm, out_hbm, tabf, tabi, bitsv, outv,
             sem_in, sem_out):
    wid = lax.axis_index("s") * _NC + lax.axis_index("c")
    pltpu.sync_copy(tabf_hbm, tabf)
    pltpu.sync_copy(tabi_hbm, tabi)

    def fire_in(s, half):
        base = wid * _EPW + s * _CH
        hs = []
        for k in range(12):
            src = tb_hbm.at[k, pl.ds(pl.multiple_of(base, 8), _CH)]
            dst = bitsv.at[pl.ds(half * (12 * _CH) + k * _CH, _CH)]
            hs.append(pltpu.async_copy(src, dst, sem_in.at[half]))
        return hs

    def fire_out(s, half):
        tb0 = (wid * _EPW + s * _CH) // 128
        hs = []
        for n in range(4):
            src = outv.at[pl.ds(half * (32 * _CH) + n * (8 * _CH), 8 * _CH)]
            dst = out_hbm.at[pl.ds(
                pl.multiple_of(n * (_NBT * 1024) + tb0 * 1024, 8), 8 * _CH)]
            hs.append(pltpu.async_copy(src, dst, sem_out.at[half]))
        return hs

    in_h = {0: fire_in(0, 0)}
    out_h = {}
    for s in range(_NSUB):
        half = s % 2
        for h in in_h.pop(s):
            h.wait()
        if s + 1 < _NSUB:
            in_h[s + 1] = fire_in(s + 1, 1 - half)
        if s - 2 in out_h:
            for h in out_h.pop(s - 2):
                h.wait()
        bb = half * (12 * _CH)
        ob0 = half * (32 * _CH)

        @plsc.parallel_loop(0, _CH // _L, 1, unroll=4)
        def vec_body(v):
            e = v * _L
            bs = [bitsv[pl.ds(bb + k * _CH + e, _L)] for k in range(12)]
            pairs = [bs[k] * 2 + bs[k + 1] for k in range(0, 12, 2)]
            quads = [pairs[i] * 4 + pairs[i + 1] for i in range(0, 6, 2)]
            addr = (quads[0] * 16 + quads[1]) * 16 + quads[2]
            ob = (v // 8) * 1024 + (v % 8) * _L
            for n in range(4):
                size = _PT_SIZE[n]
                an = jnp.bitwise_and(addr, size - 1) if n < 3 else addr
                pt_base = _PT_OFF[n] + an
                t0 = plsc.load_gather(tabf, [pt_base])
                t1 = plsc.load_gather(tabf, [pt_base + size])
                t2 = plsc.load_gather(tabf, [pt_base + 2 * size])
                h0 = (t0 > 0.5).astype(jnp.int32)
                h1 = (t1 > 0.5).astype(jnp.int32)
                h2 = (t2 > 0.5).astype(jnp.int32)
                hard = h0 * 4 + h1 * 2 + h2
                nb = ob0 + n * (8 * _CH) + ob
                outv[pl.ds(nb, _L)] = t0
                outv[pl.ds(nb + 128, _L)] = t1
                outv[pl.ds(nb + 256, _L)] = t2
                if n < 3:
                    ap = _POS_OFF[n] + an * 8 + hard
                    psize = _POS_SIZE[n]
                    for j in range(5):
                        pj = plsc.load_gather(tabf, [ap + j * psize])
                        outv[pl.ds(nb + (3 + j) * 128, _L)] = pj
                else:
                    for j in range(5):
                        av = plsc.load_gather(tabi, [an + j * 4096])
                        hv = plsc.load_gather(tabi, [_H_OFF + j * 8 + hard])
                        pj = plsc.load_gather(tabf, [_POS_OFF[3] + av + hv])
                        outv[pl.ds(nb + (3 + j) * 128, _L)] = pj

        out_h[s] = fire_out(s, half)

    for s in sorted(out_h):
        for h in out_h[s]:
            h.wait()


@functools.cache
def _build_sc_forward():
    mesh = plsc.VectorSubcoreMesh(
        core_axis_name="c", subcore_axis_name="s",
        num_cores=_NC, num_subcores=_NS)
    return pl.kernel(
        _sc_body,
        out_type=jax.ShapeDtypeStruct((4 * _NBT * 1024,), jnp.float32),
        mesh=mesh,
        scratch_types=[
            pltpu.VMEM((_TABF_LEN,), jnp.float32),
            pltpu.VMEM((_TABI_LEN,), jnp.int32),
            pltpu.VMEM((2 * _CH * 12,), jnp.int32),
            pltpu.VMEM((2 * _CH * 32,), jnp.float32),
            pltpu.SemaphoreType.DMA((2,)),
            pltpu.SemaphoreType.DMA((2,)),
        ],
        compiler_params=pltpu.CompilerParams(needs_layout_passes=False,
                                             use_tc_tiling_on_sc=True),
    )


def _aux_tables(conn4):
    c = conn4.astype(jnp.int32)
    w = (jnp.int32(1) << (11 - jnp.arange(12, dtype=jnp.int32)))
    ai = jnp.arange(4096, dtype=jnp.int32)
    sa = jnp.clip(11 - c, 0, 31)
    bits_a = (ai[None, None, :] >> sa[:, :, None]) & 1
    a_tab = jnp.sum(
        jnp.where((c < 12)[:, :, None], bits_a, 0) * w[None, :, None], axis=1)
    a_tab = a_tab.astype(jnp.int32) + (jnp.arange(5, dtype=jnp.int32) * 4096)[:, None]
    hi = jnp.arange(8, dtype=jnp.int32)
    sh = jnp.clip(14 - c, 0, 31)
    bits_h = (hi[None, None, :] >> sh[:, :, None]) & 1
    h_tab = jnp.sum(
        jnp.where((c >= 12)[:, :, None], bits_h, 0) * w[None, :, None],
        axis=1).astype(jnp.int32)
    return a_tab, h_tab


def kernel(type_bits, pattern_table_1, pattern_table_2, pattern_table_3,
           pattern_table_4, position_table_1, position_table_2,
           position_table_3, position_table_4, conn4):
    assert type_bits.shape == (_B, 12)
    pts = [pattern_table_1, pattern_table_2, pattern_table_3, pattern_table_4]
    poss = [position_table_1, position_table_2, position_table_3,
            position_table_4]
    tabf = jnp.concatenate([p.reshape(-1) for p in pts]
                           + [p.reshape(-1) for p in poss])
    a_tab, h_tab = _aux_tables(conn4)
    tabi = jnp.concatenate([a_tab.reshape(-1), h_tab.reshape(-1)])
    tbp = jnp.transpose(type_bits)
    out = _build_sc_forward()(tbp, tabf, tabi)
    return (out.reshape(4, _NBT, 8, 128)
            .transpose(1, 3, 0, 2).reshape(_B, 4, 8))


# speedup vs baseline: 1.4793x; 1.0308x over previous
import functools

import jax
import jax.numpy as jnp
from jax import lax
from jax.experimental import pallas as pl
from jax.experimental.pallas import tpu as pltpu
from jax.experimental.pallas import tpu_sc as plsc

_B = 262144
_NC, _NS, _L = 2, 16, 16
_NW = _NC * _NS
_EPW = _B // _NW
_CH = 512
_NSUB = _EPW // _CH
_NBT = _B // 128

_PT_SIZE = (8, 64, 512, 4096)
_POS_SIZE = (64, 512, 4096, 4096)
_PT_OFF = []
_POS_OFF = []
_off = 0
for _n in range(4):
    _PT_OFF.append(_off)
    _off += 3 * _PT_SIZE[_n]
for _n in range(4):
    _POS_OFF.append(_off)
    _off += 5 * _POS_SIZE[_n]
_TABF_LEN = _off
_H_OFF = 5 * 4096
_TABI_LEN = _H_OFF + 5 * 8


def _sc_body(tb_hbm, p1, p2, p3, p4, q1, q2, q3, q4, tabi_hbm,
             out_hbm, tabf, tabi, bitsv, outv, sem_in, sem_out):
    wid = lax.axis_index("s") * _NC + lax.axis_index("c")
    stage = []
    for i, src in enumerate((p1, p2, p3, p4, q1, q2, q3, q4)):
        off = (_PT_OFF + _POS_OFF)[i]
        ln = (_PT_OFF + _POS_OFF + [_TABF_LEN])[i + 1] - off
        stage.append(pltpu.async_copy(
            src, tabf.at[pl.ds(off, ln)], sem_out.at[0]))
    stage.append(pltpu.async_copy(tabi_hbm, tabi, sem_out.at[1]))
    for h in stage:
        h.wait()

    def fire_in(s, half):
        base = wid * _EPW + s * _CH
        hs = []
        for k in range(12):
            src = tb_hbm.at[k, pl.ds(pl.multiple_of(base, 8), _CH)]
            dst = bitsv.at[pl.ds(half * (12 * _CH) + k * _CH, _CH)]
            hs.append(pltpu.async_copy(src, dst, sem_in.at[half]))
        return hs

    def fire_out(s, half):
        tb0 = (wid * _EPW + s * _CH) // 128
        hs = []
        for n in range(4):
            src = outv.at[pl.ds(half * (32 * _CH) + n * (8 * _CH), 8 * _CH)]
            dst = out_hbm.at[pl.ds(
                pl.multiple_of(n * (_NBT * 1024) + tb0 * 1024, 8), 8 * _CH)]
            hs.append(pltpu.async_copy(src, dst, sem_out.at[half]))
        return hs

    in_h = {0: fire_in(0, 0)}
    out_h = {}
    for s in range(_NSUB):
        half = s % 2
        for h in in_h.pop(s):
            h.wait()
        if s + 1 < _NSUB:
            in_h[s + 1] = fire_in(s + 1, 1 - half)
        if s - 2 in out_h:
            for h in out_h.pop(s - 2):
                h.wait()
        bb = half * (12 * _CH)
        ob0 = half * (32 * _CH)

        @plsc.parallel_loop(0, _CH // _L, 1, unroll=4)
        def vec_body(v):
            e = v * _L
            bs = [bitsv[pl.ds(bb + k * _CH + e, _L)] for k in range(12)]
            pairs = [bs[k] * 2 + bs[k + 1] for k in range(0, 12, 2)]
            quads = [pairs[i] * 4 + pairs[i + 1] for i in range(0, 6, 2)]
            addr = (quads[0] * 16 + quads[1]) * 16 + quads[2]
            ob = (v // 8) * 1024 + (v % 8) * _L
            for n in range(4):
                size = _PT_SIZE[n]
                an = jnp.bitwise_and(addr, size - 1) if n < 3 else addr
                pt_base = _PT_OFF[n] + an
                t0 = plsc.load_gather(tabf, [pt_base])
                t1 = plsc.load_gather(tabf, [pt_base + size])
                t2 = plsc.load_gather(tabf, [pt_base + 2 * size])
                h0 = (t0 > 0.5).astype(jnp.int32)
                h1 = (t1 > 0.5).astype(jnp.int32)
                h2 = (t2 > 0.5).astype(jnp.int32)
                hard = h0 * 4 + h1 * 2 + h2
                nb = ob0 + n * (8 * _CH) + ob
                outv[pl.ds(nb, _L)] = t0
                outv[pl.ds(nb + 128, _L)] = t1
                outv[pl.ds(nb + 256, _L)] = t2
                if n < 3:
                    ap = _POS_OFF[n] + an * 8 + hard
                    psize = _POS_SIZE[n]
                    for j in range(5):
                        pj = plsc.load_gather(tabf, [ap + j * psize])
                        outv[pl.ds(nb + (3 + j) * 128, _L)] = pj
                else:
                    for j in range(5):
                        av = plsc.load_gather(tabi, [an + j * 4096])
                        hv = plsc.load_gather(tabi, [_H_OFF + j * 8 + hard])
                        pj = plsc.load_gather(tabf, [_POS_OFF[3] + av + hv])
                        outv[pl.ds(nb + (3 + j) * 128, _L)] = pj

        out_h[s] = fire_out(s, half)

    for s in sorted(out_h):
        for h in out_h[s]:
            h.wait()


@functools.cache
def _build_sc_forward():
    mesh = plsc.VectorSubcoreMesh(
        core_axis_name="c", subcore_axis_name="s",
        num_cores=_NC, num_subcores=_NS)
    return pl.kernel(
        _sc_body,
        out_type=jax.ShapeDtypeStruct((4 * _NBT * 1024,), jnp.float32),
        mesh=mesh,
        scratch_types=[
            pltpu.VMEM((_TABF_LEN,), jnp.float32),
            pltpu.VMEM((_TABI_LEN,), jnp.int32),
            pltpu.VMEM((2 * _CH * 12,), jnp.int32),
            pltpu.VMEM((2 * _CH * 32,), jnp.float32),
            pltpu.SemaphoreType.DMA((2,)),
            pltpu.SemaphoreType.DMA((2,)),
        ],
        compiler_params=pltpu.CompilerParams(needs_layout_passes=False,
                                             use_tc_tiling_on_sc=True),
    )


def _aux_tables(conn4):
    c = conn4.astype(jnp.int32)
    w = (jnp.int32(1) << (11 - jnp.arange(12, dtype=jnp.int32)))
    ai = jnp.arange(4096, dtype=jnp.int32)
    sa = jnp.clip(11 - c, 0, 31)
    bits_a = (ai[None, None, :] >> sa[:, :, None]) & 1
    a_tab = jnp.sum(
        jnp.where((c < 12)[:, :, None], bits_a, 0) * w[None, :, None], axis=1)
    a_tab = a_tab.astype(jnp.int32) + (jnp.arange(5, dtype=jnp.int32) * 4096)[:, None]
    hi = jnp.arange(8, dtype=jnp.int32)
    sh = jnp.clip(14 - c, 0, 31)
    bits_h = (hi[None, None, :] >> sh[:, :, None]) & 1
    h_tab = jnp.sum(
        jnp.where((c >= 12)[:, :, None], bits_h, 0) * w[None, :, None],
        axis=1).astype(jnp.int32)
    return a_tab, h_tab


def kernel(type_bits, pattern_table_1, pattern_table_2, pattern_table_3,
           pattern_table_4, position_table_1, position_table_2,
           position_table_3, position_table_4, conn4):
    assert type_bits.shape == (_B, 12)
    pts = [pattern_table_1, pattern_table_2, pattern_table_3, pattern_table_4]
    poss = [position_table_1, position_table_2, position_table_3,
            position_table_4]
    a_tab, h_tab = _aux_tables(conn4)
    tabi = jnp.concatenate([a_tab.reshape(-1), h_tab.reshape(-1)])
    tbp = jnp.transpose(type_bits)
    out = _build_sc_forward()(
        tbp, *[p.reshape(-1) for p in pts], *[p.reshape(-1) for p in poss],
        tabi)
    return (out.reshape(4, _NBT, 8, 128)
            .transpose(1, 3, 0, 2).reshape(_B, 4, 8))
